# Initial kernel scaffold; baseline (speedup 1.0000x reference)
import jax
import jax.numpy as jnp
from jax.experimental import pallas as pl


def kernel(x, edge_index, edge_type, params):
    raise NotImplementedError("write your pallas kernel here")


# speedup vs baseline: 7.5485x; 7.5485x over previous
import jax
import jax.numpy as jnp
from jax import lax
from jax.experimental import pallas as pl
from jax.experimental.pallas import tpu as pltpu
from jax.experimental.pallas import tpu_sc as plsc

N = 10000
E = 320000
H = 128
R = 8
L = 3
NC = 2
NS = 16
LANES = 16
RPC = R // NC
EPT = E // NS
NVEC = EPT // LANES
CH = 128
PADQ = 256
REG = 22528
CBUF = 20608
CHM = 400
NCHM = EPT // CHM
NPAD = 10240
RPT = NPAD // NS
DUMMY = 10200
ZR = 64
BN = 1000
GRID = N // BN

_mesh = plsc.VectorSubcoreMesh(
    core_axis_name="c", subcore_axis_name="s", num_cores=NC, num_subcores=NS
)


def _bin_body(src_hbm, dst_hbm, et_hbm,
              bsrc_out, bdst_out, plen_out,
              src_v, dst_v, et_v, csrc_v, cdst_v, plen_v):
    c = lax.axis_index("c")
    s = lax.axis_index("s")
    rbase = (c * NS + s) * REG
    base = s * EPT
    iota = lax.iota(jnp.int32, LANES)
    plvec = jnp.zeros((LANES,), jnp.int32)
    off = jnp.int32(0)
    for p in range(RPC):
        rel = c * RPC + p

        def scan_chunk(kc, cnt0):
            mb = pl.ds(pl.multiple_of(base + kc * CHM, 8), CHM)
            pltpu.sync_copy(src_hbm.at[mb], src_v)
            pltpu.sync_copy(dst_hbm.at[mb], dst_v)
            pltpu.sync_copy(et_hbm.at[mb], et_v)

            def scan(i, cnt):
                sl = pl.ds(i * LANES, LANES)
                m = et_v[sl] == rel
                mi = m.astype(jnp.int32)
                pos = cnt + lax.cumsum(mi, axis=0) - mi
                plsc.store_scatter(csrc_v, [pos], src_v[sl], mask=m)
                plsc.store_scatter(cdst_v, [pos], dst_v[sl], mask=m)
                return cnt + jnp.sum(mi)

            return lax.fori_loop(0, CHM // LANES, scan, cnt0)

        cnt = lax.fori_loop(0, NCHM, scan_chunk, jnp.int32(0))
        for j in range(PADQ // LANES):
            pidx = cnt + j * LANES + iota
            plsc.store_scatter(csrc_v, [pidx], jnp.zeros((LANES,), jnp.int32))
            plsc.store_scatter(cdst_v, [pidx],
                               jnp.full((LANES,), DUMMY, jnp.int32))
        padded = ((cnt + PADQ - 1) // PADQ) * PADQ
        nout = (padded + 1023) // 1024

        def flush(k, o):
            sl_v = pl.ds(k * 1024, 1024)
            sl_h = pl.ds(pl.multiple_of(rbase + o + k * 1024, 256), 1024)
            pltpu.sync_copy(csrc_v.at[sl_v], bsrc_out.at[sl_h])
            pltpu.sync_copy(cdst_v.at[sl_v], bdst_out.at[sl_h])
            return o

        lax.fori_loop(0, nout, flush, off)
        plvec = plvec + jnp.where(iota == p, padded, 0)
        off = off + padded
    plen_v[...] = plvec
    pltpu.sync_copy(
        plen_v,
        plen_out.at[pl.ds(pl.multiple_of((c * NS + s) * LANES, LANES), LANES)])


def _bin_edges(src, dst, et):
    f = pl.kernel(
        _bin_body,
        out_type=(
            jax.ShapeDtypeStruct((NC * NS * REG,), jnp.int32),
            jax.ShapeDtypeStruct((NC * NS * REG,), jnp.int32),
            jax.ShapeDtypeStruct((NC * NS * LANES,), jnp.int32),
        ),
        mesh=_mesh,
        scratch_types=[
            pltpu.VMEM((CHM,), jnp.int32),
            pltpu.VMEM((CHM,), jnp.int32),
            pltpu.VMEM((CHM,), jnp.int32),
            pltpu.VMEM((CBUF,), jnp.int32),
            pltpu.VMEM((CBUF,), jnp.int32),
            pltpu.VMEM((LANES,), jnp.int32),
        ],
        compiler_params=pltpu.CompilerParams(needs_layout_passes=False),
    )
    return f(src, dst, et)


def _cnt_body(bdst_hbm, plens_hbm, ones_hbm, zcnt_hbm, cnt_out,
              acc_c, idx_d, ones_v, zcnt_v, plen_v):
    c = lax.axis_index("c")
    s = lax.axis_index("s")
    rbase = (c * NS + s) * REG
    iota = lax.iota(jnp.int32, LANES)
    pltpu.sync_copy(
        plens_hbm.at[pl.ds(pl.multiple_of((c * NS + s) * LANES, LANES), LANES)],
        plen_v)
    pltpu.sync_copy(ones_hbm, ones_v)
    pltpu.sync_copy(zcnt_hbm, zcnt_v)
    plvec = plen_v[...]
    row0 = pl.multiple_of(s * RPT, RPT)
    for p in range(RPC):
        for k in range(RPT // ZR):
            sl_z = pl.ds(pl.multiple_of(row0 + k * ZR, ZR), ZR)
            pltpu.sync_copy(zcnt_v, acc_c.at[sl_z])
        plsc.subcore_barrier()
        lp = jnp.max(jnp.where(iota == p, plvec, 0))
        off = jnp.sum(jnp.where(iota < p, plvec, 0))
        nch = lp // CH

        def chunk(k, carry):
            so = pl.ds(pl.multiple_of(rbase + off + k * CH, CH), CH)
            pltpu.sync_copy(bdst_hbm.at[so], idx_d)
            pltpu.sync_copy(ones_v, acc_c.at[idx_d], add=True)
            return carry

        lax.fori_loop(0, nch, chunk, 0)
        plsc.subcore_barrier()
        obase = pl.ds(pl.multiple_of((c * RPC + p) * NPAD + row0, ZR), RPT)
        pltpu.sync_copy(acc_c.at[pl.ds(row0, RPT)], cnt_out.at[obase])


def _cnt_edges(bdst, plens, ones, zcnt):
    f = pl.kernel(
        _cnt_body,
        out_type=jax.ShapeDtypeStruct((R * NPAD, H), jnp.float32),
        mesh=_mesh,
        scratch_types=[
            pltpu.VMEM_SHARED((NPAD, H), jnp.float32),
            pltpu.VMEM((CH,), jnp.int32),
            pltpu.VMEM((CH, H), jnp.float32),
            pltpu.VMEM((ZR, H), jnp.float32),
            pltpu.VMEM((LANES,), jnp.int32),
        ],
        compiler_params=pltpu.CompilerParams(needs_layout_passes=False),
    )
    return f(bdst, plens, ones, zcnt)


def _seg_body(h_hbm, bsrc_hbm, bdst_hbm, plens_hbm, zero_hbm,
              sh_out,
              acc_h, idx_s, idx_d, rows, zero_v, plen_v, sem):
    c = lax.axis_index("c")
    s = lax.axis_index("s")
    rbase = (c * NS + s) * REG
    iota = lax.iota(jnp.int32, LANES)
    pltpu.sync_copy(
        plens_hbm.at[pl.ds(pl.multiple_of((c * NS + s) * LANES, LANES), LANES)],
        plen_v)
    pltpu.sync_copy(zero_hbm, zero_v)
    plvec = plen_v[...]
    row0 = pl.multiple_of(s * RPT, RPT)
    for p in range(RPC):
        for k in range(RPT // ZR):
            sl_z = pl.ds(pl.multiple_of(row0 + k * ZR, ZR), ZR)
            pltpu.sync_copy(zero_v, acc_h.at[sl_z])
        plsc.subcore_barrier()
        lp = jnp.max(jnp.where(iota == p, plvec, 0))
        off = jnp.sum(jnp.where(iota < p, plvec, 0))
        nch = lp // CH

        def chunk(k, carry):
            so = pl.ds(pl.multiple_of(rbase + off + k * CH, CH), CH)
            pltpu.sync_copy(bsrc_hbm.at[so], idx_s)
            pltpu.sync_copy(bdst_hbm.at[so], idx_d)
            pltpu.async_copy(h_hbm.at[idx_s], rows, sem).wait()
            pltpu.sync_copy(rows, acc_h.at[idx_d], add=True)
            return carry

        lax.fori_loop(0, nch, chunk, 0)
        plsc.subcore_barrier()
        obase = pl.ds(pl.multiple_of((c * RPC + p) * NPAD + row0, ZR), RPT)
        pltpu.sync_copy(acc_h.at[pl.ds(row0, RPT)], sh_out.at[obase])


def _seg_sum(h, bsrc, bdst, plens, zero):
    f = pl.kernel(
        _seg_body,
        out_type=jax.ShapeDtypeStruct((R * NPAD, H), jnp.float32),
        mesh=_mesh,
        scratch_types=[
            pltpu.VMEM_SHARED((NPAD, H), jnp.float32),
            pltpu.VMEM((CH,), jnp.int32),
            pltpu.VMEM((CH,), jnp.int32),
            pltpu.VMEM((CH, H), jnp.float32),
            pltpu.VMEM((ZR, H), jnp.float32),
            pltpu.VMEM((LANES,), jnp.int32),
            pltpu.SemaphoreType.DMA,
        ],
        compiler_params=pltpu.CompilerParams(needs_layout_passes=False),
    )
    return f(h, bsrc, bdst, plens, zero)


def _matmul_body(x_ref, w_ref, b_ref, o_ref):
    o_ref[...] = (
        jnp.dot(x_ref[...], w_ref[...], preferred_element_type=jnp.float32)
        + b_ref[...]
    )


def _in_proj(x, w, b):
    return pl.pallas_call(
        _matmul_body,
        grid=(GRID,),
        in_specs=[
            pl.BlockSpec((BN, H), lambda i: (i, 0)),
            pl.BlockSpec((H, H), lambda i: (0, 0)),
            pl.BlockSpec((1, H), lambda i: (0, 0)),
        ],
        out_specs=pl.BlockSpec((BN, H), lambda i: (i, 0)),
        out_shape=jax.ShapeDtypeStruct((N, H), jnp.float32),
    )(x, w, b)


def _dense_body(sh_ref, scnt_ref, h_ref, w_ref, wroot_ref, b_ref,
                hpre_ref, stats_ref, acc_ref):
    i = pl.program_id(0)
    agg = jnp.zeros((BN, H), jnp.float32)
    for r in range(R):
        cnt = scnt_ref[:, r:r + 1]
        rc = 1.0 / jnp.maximum(cnt, 1.0)
        agg = agg + jnp.dot(sh_ref[r] * rc, w_ref[r],
                            preferred_element_type=jnp.float32)
    agg = agg + jnp.dot(h_ref[...], wroot_ref[...],
                        preferred_element_type=jnp.float32) + b_ref[...]

    @pl.when(i == 0)
    def _():
        acc_ref[...] = jnp.zeros((2, H), jnp.float32)

    colsum = jnp.sum(agg, axis=0)[None, :]
    colsq = jnp.sum(agg * agg, axis=0)[None, :]
    acc_ref[...] = acc_ref[...] + jnp.concatenate([colsum, colsq], axis=0)

    @pl.when(i == pl.num_programs(0) - 1)
    def _():
        stats_ref[...] = acc_ref[...]

    hpre_ref[...] = agg


def _dense(sh, scnt, h, w, wroot, b):
    return pl.pallas_call(
        _dense_body,
        grid=(GRID,),
        in_specs=[
            pl.BlockSpec((R, BN, H), lambda i: (0, i, 0)),
            pl.BlockSpec((BN, R), lambda i: (i, 0)),
            pl.BlockSpec((BN, H), lambda i: (i, 0)),
            pl.BlockSpec((R, H, H), lambda i: (0, 0, 0)),
            pl.BlockSpec((H, H), lambda i: (0, 0)),
            pl.BlockSpec((1, H), lambda i: (0, 0)),
        ],
        out_specs=[
            pl.BlockSpec((BN, H), lambda i: (i, 0)),
            pl.BlockSpec((2, H), lambda i: (0, 0)),
        ],
        out_shape=[
            jax.ShapeDtypeStruct((N, H), jnp.float32),
            jax.ShapeDtypeStruct((2, H), jnp.float32),
        ],
        scratch_shapes=[pltpu.VMEM((2, H), jnp.float32)],
    )(sh, scnt, h, w, wroot, b)


def _bn_body(hpre_ref, stats_ref, g_ref, b_ref, o_ref):
    mu = stats_ref[0:1, :] * (1.0 / N)
    ex2 = stats_ref[1:2, :] * (1.0 / N)
    var = ex2 - mu * mu
    inv = lax.rsqrt(var + 1e-5)
    y = g_ref[...] * (hpre_ref[...] - mu) * inv + b_ref[...]
    o_ref[...] = jnp.where(y >= 0.0, y, 0.1 * y)


def _bn_apply(hpre, stats, g, b):
    return pl.pallas_call(
        _bn_body,
        grid=(GRID,),
        in_specs=[
            pl.BlockSpec((BN, H), lambda i: (i, 0)),
            pl.BlockSpec((2, H), lambda i: (0, 0)),
            pl.BlockSpec((1, H), lambda i: (0, 0)),
            pl.BlockSpec((1, H), lambda i: (0, 0)),
        ],
        out_specs=pl.BlockSpec((BN, H), lambda i: (i, 0)),
        out_shape=jax.ShapeDtypeStruct((N, H), jnp.float32),
    )(hpre, stats, g, b)


def _fuse_body(h1_ref, h2_ref, h3_ref, w1_ref, b1_ref, w2_ref, b2_ref,
               ow_ref, ob_ref, o_ref):
    hs = (h1_ref[...], h2_ref[...], h3_ref[...])
    ss = []
    for hr in hs:
        t = jnp.maximum(
            jnp.dot(hr, w1_ref[...], preferred_element_type=jnp.float32)
            + b1_ref[...], 0.0)
        ss.append(jnp.sum(t * w2_ref[...], axis=1, keepdims=True)
                  + b2_ref[...])
    m = jnp.maximum(jnp.maximum(ss[0], ss[1]), ss[2])
    es = [jnp.exp(s - m) for s in ss]
    z = es[0] + es[1] + es[2]
    ms = (es[0] * hs[0] + es[1] * hs[1] + es[2] * hs[2]) / z
    out = jnp.dot(ms, ow_ref[...], preferred_element_type=jnp.float32) \
        + ob_ref[...]
    nrm = jnp.sqrt(jnp.sum(out * out, axis=1, keepdims=True))
    o_ref[...] = out / jnp.maximum(nrm, 1e-12)


def _fuse(h1, h2, h3, w1, b1, w2t, b2, ow, ob):
    hq = H // 4
    return pl.pallas_call(
        _fuse_body,
        grid=(GRID,),
        in_specs=[
            pl.BlockSpec((BN, H), lambda i: (i, 0)),
            pl.BlockSpec((BN, H), lambda i: (i, 0)),
            pl.BlockSpec((BN, H), lambda i: (i, 0)),
            pl.BlockSpec((H, hq), lambda i: (0, 0)),
            pl.BlockSpec((1, hq), lambda i: (0, 0)),
            pl.BlockSpec((1, hq), lambda i: (0, 0)),
            pl.BlockSpec((1, 1), lambda i: (0, 0)),
            pl.BlockSpec((H, H), lambda i: (0, 0)),
            pl.BlockSpec((1, H), lambda i: (0, 0)),
        ],
        out_specs=pl.BlockSpec((BN, H), lambda i: (i, 0)),
        out_shape=jax.ShapeDtypeStruct((N, H), jnp.float32),
    )(h1, h2, h3, w1, b1, w2t, b2, ow, ob)


def kernel(x, edge_index, edge_type, params):
    src = edge_index[0]
    dst = edge_index[1]
    et = edge_type.astype(jnp.int32)

    zero = jnp.zeros((ZR, H), jnp.float32)
    ones = jnp.ones((CH, H), jnp.float32)

    bsrc, bdst, plens = _bin_edges(src, dst, et)
    cnt = _cnt_edges(bdst, plens, ones, zero)
    scnt = cnt.reshape(R, NPAD, H)[:, :, 0].T

    h = _in_proj(x, params['in_w'], params['in_b'].reshape(1, H))
    reps = []
    for i in range(L):
        sh = _seg_sum(h, bsrc, bdst, plens, zero)
        sh = sh.reshape(R, NPAD, H)
        hpre, stats = _dense(sh, scnt, h, params['conv_w'][i],
                             params['conv_root'][i],
                             params['conv_b'][i].reshape(1, H))
        h = _bn_apply(hpre, stats, params['bn_g'][i].reshape(1, H),
                      params['bn_b'][i].reshape(1, H))
        reps.append(h)

    out = _fuse(
        reps[0], reps[1], reps[2],
        params['att_w1'], params['att_b1'].reshape(1, H // 4),
        params['att_w2'].reshape(1, H // 4), params['att_b2'].reshape(1, 1),
        params['out_w'], params['out_b'].reshape(1, H),
    )
    return out


# speedup vs baseline: 8.3086x; 1.1007x over previous
import jax
import jax.numpy as jnp
from jax import lax
from jax.experimental import pallas as pl
from jax.experimental.pallas import tpu as pltpu
from jax.experimental.pallas import tpu_sc as plsc

N = 10000
E = 320000
H = 128
R = 8
L = 3
NC = 2
NS = 16
LANES = 16
RPC = R // NC
EPT = E // NS
NVEC = EPT // LANES
CH = 128
PADQ = 256
REG = 22528
CBUF = 20608
CHM = 400
NCHM = EPT // CHM
NPAD = 10240
RPT = NPAD // NS
DUMMY = 10200
ZR = 64
BN = 1000
GRID = N // BN

_mesh = plsc.VectorSubcoreMesh(
    core_axis_name="c", subcore_axis_name="s", num_cores=NC, num_subcores=NS
)


def _bin_body(src_hbm, dst_hbm, et_hbm,
              bsrc_out, bdst_out, plen_out,
              src_v, dst_v, et_v, csrc_v, cdst_v, plen_v):
    c = lax.axis_index("c")
    s = lax.axis_index("s")
    rbase = (c * NS + s) * REG
    base = s * EPT
    iota = lax.iota(jnp.int32, LANES)
    plvec = jnp.zeros((LANES,), jnp.int32)
    off = jnp.int32(0)
    for p in range(RPC):
        rel = c * RPC + p

        def scan_chunk(kc, cnt0):
            mb = pl.ds(pl.multiple_of(base + kc * CHM, 8), CHM)
            pltpu.sync_copy(src_hbm.at[mb], src_v)
            pltpu.sync_copy(dst_hbm.at[mb], dst_v)
            pltpu.sync_copy(et_hbm.at[mb], et_v)

            def scan(i, cnt):
                sl = pl.ds(i * LANES, LANES)
                m = et_v[sl] == rel
                mi = m.astype(jnp.int32)
                pos = cnt + lax.cumsum(mi, axis=0) - mi
                plsc.store_scatter(csrc_v, [pos], src_v[sl], mask=m)
                plsc.store_scatter(cdst_v, [pos], dst_v[sl], mask=m)
                return cnt + jnp.sum(mi)

            return lax.fori_loop(0, CHM // LANES, scan, cnt0)

        cnt = lax.fori_loop(0, NCHM, scan_chunk, jnp.int32(0))
        for j in range(PADQ // LANES):
            pidx = cnt + j * LANES + iota
            plsc.store_scatter(csrc_v, [pidx], jnp.zeros((LANES,), jnp.int32))
            plsc.store_scatter(cdst_v, [pidx],
                               jnp.full((LANES,), DUMMY, jnp.int32))
        padded = ((cnt + PADQ - 1) // PADQ) * PADQ
        nout = (padded + 1023) // 1024

        def flush(k, o):
            sl_v = pl.ds(k * 1024, 1024)
            sl_h = pl.ds(pl.multiple_of(rbase + o + k * 1024, 256), 1024)
            pltpu.sync_copy(csrc_v.at[sl_v], bsrc_out.at[sl_h])
            pltpu.sync_copy(cdst_v.at[sl_v], bdst_out.at[sl_h])
            return o

        lax.fori_loop(0, nout, flush, off)
        plvec = plvec + jnp.where(iota == p, padded, 0)
        off = off + padded
    plen_v[...] = plvec
    pltpu.sync_copy(
        plen_v,
        plen_out.at[pl.ds(pl.multiple_of((c * NS + s) * LANES, LANES), LANES)])


def _bin_edges(src, dst, et):
    f = pl.kernel(
        _bin_body,
        out_type=(
            jax.ShapeDtypeStruct((NC * NS * REG,), jnp.int32),
            jax.ShapeDtypeStruct((NC * NS * REG,), jnp.int32),
            jax.ShapeDtypeStruct((NC * NS * LANES,), jnp.int32),
        ),
        mesh=_mesh,
        scratch_types=[
            pltpu.VMEM((CHM,), jnp.int32),
            pltpu.VMEM((CHM,), jnp.int32),
            pltpu.VMEM((CHM,), jnp.int32),
            pltpu.VMEM((CBUF,), jnp.int32),
            pltpu.VMEM((CBUF,), jnp.int32),
            pltpu.VMEM((LANES,), jnp.int32),
        ],
        compiler_params=pltpu.CompilerParams(needs_layout_passes=False),
    )
    return f(src, dst, et)


def _cnt_body(bdst_hbm, plens_hbm, ones_hbm, zcnt_hbm, cnt_out,
              acc_c, idx_d, ones_v, zcnt_v, plen_v):
    c = lax.axis_index("c")
    s = lax.axis_index("s")
    rbase = (c * NS + s) * REG
    iota = lax.iota(jnp.int32, LANES)
    pltpu.sync_copy(
        plens_hbm.at[pl.ds(pl.multiple_of((c * NS + s) * LANES, LANES), LANES)],
        plen_v)
    pltpu.sync_copy(ones_hbm, ones_v)
    pltpu.sync_copy(zcnt_hbm, zcnt_v)
    plvec = plen_v[...]
    row0 = pl.multiple_of(s * RPT, RPT)
    for p in range(RPC):
        for k in range(RPT // ZR):
            sl_z = pl.ds(pl.multiple_of(row0 + k * ZR, ZR), ZR)
            pltpu.sync_copy(zcnt_v, acc_c.at[sl_z])
        plsc.subcore_barrier()
        lp = jnp.max(jnp.where(iota == p, plvec, 0))
        off = jnp.sum(jnp.where(iota < p, plvec, 0))
        nch = lp // CH

        def chunk(k, carry):
            so = pl.ds(pl.multiple_of(rbase + off + k * CH, CH), CH)
            pltpu.sync_copy(bdst_hbm.at[so], idx_d)
            pltpu.sync_copy(ones_v, acc_c.at[idx_d], add=True)
            return carry

        lax.fori_loop(0, nch, chunk, 0)
        plsc.subcore_barrier()
        obase = pl.ds(pl.multiple_of((c * RPC + p) * NPAD + row0, ZR), RPT)
        pltpu.sync_copy(acc_c.at[pl.ds(row0, RPT)], cnt_out.at[obase])


def _cnt_edges(bdst, plens, ones, zcnt):
    f = pl.kernel(
        _cnt_body,
        out_type=jax.ShapeDtypeStruct((R * NPAD, H), jnp.float32),
        mesh=_mesh,
        scratch_types=[
            pltpu.VMEM_SHARED((NPAD, H), jnp.float32),
            pltpu.VMEM((CH,), jnp.int32),
            pltpu.VMEM((CH, H), jnp.float32),
            pltpu.VMEM((ZR, H), jnp.float32),
            pltpu.VMEM((LANES,), jnp.int32),
        ],
        compiler_params=pltpu.CompilerParams(needs_layout_passes=False),
    )
    return f(bdst, plens, ones, zcnt)


def _seg_body(h_hbm, bsrc_hbm, bdst_hbm, plens_hbm, zero_hbm,
              sh_out,
              acc_h, idx_s0, idx_d0, rows0, idx_s1, idx_d1, rows1, zero_v,
              plen_v, sem0, sem1):
    c = lax.axis_index("c")
    s = lax.axis_index("s")
    rbase = (c * NS + s) * REG
    iota = lax.iota(jnp.int32, LANES)
    pltpu.sync_copy(
        plens_hbm.at[pl.ds(pl.multiple_of((c * NS + s) * LANES, LANES), LANES)],
        plen_v)
    pltpu.sync_copy(zero_hbm, zero_v)
    plvec = plen_v[...]
    row0 = pl.multiple_of(s * RPT, RPT)
    bufs = ((idx_s0, idx_d0, rows0, sem0), (idx_s1, idx_d1, rows1, sem1))

    for p in range(RPC):
        for k in range(RPT // ZR):
            sl_z = pl.ds(pl.multiple_of(row0 + k * ZR, ZR), ZR)
            pltpu.sync_copy(zero_v, acc_h.at[sl_z])
        plsc.subcore_barrier()
        lp = jnp.max(jnp.where(iota == p, plvec, 0))
        off = jnp.sum(jnp.where(iota < p, plvec, 0))
        nch = lp // CH

        def start(k, b):
            isv, idv, rv, sm = bufs[b]
            so = pl.ds(pl.multiple_of(rbase + off + k * CH, CH), CH)
            pltpu.sync_copy(bsrc_hbm.at[so], isv)
            pltpu.sync_copy(bdst_hbm.at[so], idv)
            pltpu.async_copy(h_hbm.at[isv], rv, sm)

        def finish(b):
            isv, idv, rv, sm = bufs[b]
            pltpu.make_async_copy(h_hbm.at[isv], rv, sm).wait()
            pltpu.sync_copy(rv, acc_h.at[idv], add=True)

        @pl.when(nch > 0)
        def _():
            start(0, 0)

            def body(j, carry):
                k1 = 2 * j + 1
                start(k1, 1)
                finish(0)

                @pl.when(k1 + 1 < nch)
                def _():
                    start(k1 + 1, 0)

                finish(1)
                return carry

            lax.fori_loop(0, nch // 2, body, 0)

        plsc.subcore_barrier()
        obase = pl.ds(pl.multiple_of((c * RPC + p) * NPAD + row0, ZR), RPT)
        pltpu.sync_copy(acc_h.at[pl.ds(row0, RPT)], sh_out.at[obase])


def _seg_sum(h, bsrc, bdst, plens, zero):
    f = pl.kernel(
        _seg_body,
        out_type=jax.ShapeDtypeStruct((R * NPAD, H), jnp.float32),
        mesh=_mesh,
        scratch_types=[
            pltpu.VMEM_SHARED((NPAD, H), jnp.float32),
            pltpu.VMEM((CH,), jnp.int32),
            pltpu.VMEM((CH,), jnp.int32),
            pltpu.VMEM((CH, H), jnp.float32),
            pltpu.VMEM((CH,), jnp.int32),
            pltpu.VMEM((CH,), jnp.int32),
            pltpu.VMEM((CH, H), jnp.float32),
            pltpu.VMEM((ZR, H), jnp.float32),
            pltpu.VMEM((LANES,), jnp.int32),
            pltpu.SemaphoreType.DMA,
            pltpu.SemaphoreType.DMA,
        ],
        compiler_params=pltpu.CompilerParams(needs_layout_passes=False),
    )
    return f(h, bsrc, bdst, plens, zero)


def _matmul_body(x_ref, w_ref, b_ref, o_ref):
    o_ref[...] = (
        jnp.dot(x_ref[...], w_ref[...], preferred_element_type=jnp.float32)
        + b_ref[...]
    )


def _in_proj(x, w, b):
    return pl.pallas_call(
        _matmul_body,
        grid=(GRID,),
        in_specs=[
            pl.BlockSpec((BN, H), lambda i: (i, 0)),
            pl.BlockSpec((H, H), lambda i: (0, 0)),
            pl.BlockSpec((1, H), lambda i: (0, 0)),
        ],
        out_specs=pl.BlockSpec((BN, H), lambda i: (i, 0)),
        out_shape=jax.ShapeDtypeStruct((N, H), jnp.float32),
    )(x, w, b)


def _dense_body(sh_ref, scnt_ref, h_ref, w_ref, wroot_ref, b_ref,
                hpre_ref, stats_ref, acc_ref):
    i = pl.program_id(0)
    agg = jnp.zeros((BN, H), jnp.float32)
    for r in range(R):
        cnt = scnt_ref[:, r:r + 1]
        rc = 1.0 / jnp.maximum(cnt, 1.0)
        agg = agg + jnp.dot(sh_ref[r] * rc, w_ref[r],
                            preferred_element_type=jnp.float32)
    agg = agg + jnp.dot(h_ref[...], wroot_ref[...],
                        preferred_element_type=jnp.float32) + b_ref[...]

    @pl.when(i == 0)
    def _():
        acc_ref[...] = jnp.zeros((2, H), jnp.float32)

    colsum = jnp.sum(agg, axis=0)[None, :]
    colsq = jnp.sum(agg * agg, axis=0)[None, :]
    acc_ref[...] = acc_ref[...] + jnp.concatenate([colsum, colsq], axis=0)

    @pl.when(i == pl.num_programs(0) - 1)
    def _():
        stats_ref[...] = acc_ref[...]

    hpre_ref[...] = agg


def _dense(sh, scnt, h, w, wroot, b):
    return pl.pallas_call(
        _dense_body,
        grid=(GRID,),
        in_specs=[
            pl.BlockSpec((R, BN, H), lambda i: (0, i, 0)),
            pl.BlockSpec((BN, R), lambda i: (i, 0)),
            pl.BlockSpec((BN, H), lambda i: (i, 0)),
            pl.BlockSpec((R, H, H), lambda i: (0, 0, 0)),
            pl.BlockSpec((H, H), lambda i: (0, 0)),
            pl.BlockSpec((1, H), lambda i: (0, 0)),
        ],
        out_specs=[
            pl.BlockSpec((BN, H), lambda i: (i, 0)),
            pl.BlockSpec((2, H), lambda i: (0, 0)),
        ],
        out_shape=[
            jax.ShapeDtypeStruct((N, H), jnp.float32),
            jax.ShapeDtypeStruct((2, H), jnp.float32),
        ],
        scratch_shapes=[pltpu.VMEM((2, H), jnp.float32)],
    )(sh, scnt, h, w, wroot, b)


def _bn_body(hpre_ref, stats_ref, g_ref, b_ref, o_ref):
    mu = stats_ref[0:1, :] * (1.0 / N)
    ex2 = stats_ref[1:2, :] * (1.0 / N)
    var = ex2 - mu * mu
    inv = lax.rsqrt(var + 1e-5)
    y = g_ref[...] * (hpre_ref[...] - mu) * inv + b_ref[...]
    o_ref[...] = jnp.where(y >= 0.0, y, 0.1 * y)


def _bn_apply(hpre, stats, g, b):
    return pl.pallas_call(
        _bn_body,
        grid=(GRID,),
        in_specs=[
            pl.BlockSpec((BN, H), lambda i: (i, 0)),
            pl.BlockSpec((2, H), lambda i: (0, 0)),
            pl.BlockSpec((1, H), lambda i: (0, 0)),
            pl.BlockSpec((1, H), lambda i: (0, 0)),
        ],
        out_specs=pl.BlockSpec((BN, H), lambda i: (i, 0)),
        out_shape=jax.ShapeDtypeStruct((N, H), jnp.float32),
    )(hpre, stats, g, b)


def _fuse_body(h1_ref, h2_ref, h3_ref, w1_ref, b1_ref, w2_ref, b2_ref,
               ow_ref, ob_ref, o_ref):
    hs = (h1_ref[...], h2_ref[...], h3_ref[...])
    ss = []
    for hr in hs:
        t = jnp.maximum(
            jnp.dot(hr, w1_ref[...], preferred_element_type=jnp.float32)
            + b1_ref[...], 0.0)
        ss.append(jnp.sum(t * w2_ref[...], axis=1, keepdims=True)
                  + b2_ref[...])
    m = jnp.maximum(jnp.maximum(ss[0], ss[1]), ss[2])
    es = [jnp.exp(s - m) for s in ss]
    z = es[0] + es[1] + es[2]
    ms = (es[0] * hs[0] + es[1] * hs[1] + es[2] * hs[2]) / z
    out = jnp.dot(ms, ow_ref[...], preferred_element_type=jnp.float32) \
        + ob_ref[...]
    nrm = jnp.sqrt(jnp.sum(out * out, axis=1, keepdims=True))
    o_ref[...] = out / jnp.maximum(nrm, 1e-12)


def _fuse(h1, h2, h3, w1, b1, w2t, b2, ow, ob):
    hq = H // 4
    return pl.pallas_call(
        _fuse_body,
        grid=(GRID,),
        in_specs=[
            pl.BlockSpec((BN, H), lambda i: (i, 0)),
            pl.BlockSpec((BN, H), lambda i: (i, 0)),
            pl.BlockSpec((BN, H), lambda i: (i, 0)),
            pl.BlockSpec((H, hq), lambda i: (0, 0)),
            pl.BlockSpec((1, hq), lambda i: (0, 0)),
            pl.BlockSpec((1, hq), lambda i: (0, 0)),
            pl.BlockSpec((1, 1), lambda i: (0, 0)),
            pl.BlockSpec((H, H), lambda i: (0, 0)),
            pl.BlockSpec((1, H), lambda i: (0, 0)),
        ],
        out_specs=pl.BlockSpec((BN, H), lambda i: (i, 0)),
        out_shape=jax.ShapeDtypeStruct((N, H), jnp.float32),
    )(h1, h2, h3, w1, b1, w2t, b2, ow, ob)


def kernel(x, edge_index, edge_type, params):
    src = edge_index[0]
    dst = edge_index[1]
    et = edge_type.astype(jnp.int32)

    zero = jnp.zeros((ZR, H), jnp.float32)
    ones = jnp.ones((CH, H), jnp.float32)

    bsrc, bdst, plens = _bin_edges(src, dst, et)
    cnt = _cnt_edges(bdst, plens, ones, zero)
    scnt = cnt.reshape(R, NPAD, H)[:, :, 0].T

    h = _in_proj(x, params['in_w'], params['in_b'].reshape(1, H))
    reps = []
    for i in range(L):
        sh = _seg_sum(h, bsrc, bdst, plens, zero)
        sh = sh.reshape(R, NPAD, H)
        hpre, stats = _dense(sh, scnt, h, params['conv_w'][i],
                             params['conv_root'][i],
                             params['conv_b'][i].reshape(1, H))
        h = _bn_apply(hpre, stats, params['bn_g'][i].reshape(1, H),
                      params['bn_b'][i].reshape(1, H))
        reps.append(h)

    out = _fuse(
        reps[0], reps[1], reps[2],
        params['att_w1'], params['att_b1'].reshape(1, H // 4),
        params['att_w2'].reshape(1, H // 4), params['att_b2'].reshape(1, 1),
        params['out_w'], params['out_b'].reshape(1, H),
    )
    return out


# speedup vs baseline: 8.3152x; 1.0008x over previous
import jax
import jax.numpy as jnp
from jax import lax
from jax.experimental import pallas as pl
from jax.experimental.pallas import tpu as pltpu
from jax.experimental.pallas import tpu_sc as plsc

N = 10000
E = 320000
H = 128
R = 8
L = 3
NC = 2
NS = 16
LANES = 16
RPC = R // NC
EPT = E // NS
NVEC = EPT // LANES
CH = 128
PADQ = 256
REG = 22528
CBUF = 20608
CHM = 400
NCHM = EPT // CHM
NPAD = 10240
RPT = NPAD // NS
DUMMY = 10200
ZR = 64
BN = 1000
GRID = N // BN

_mesh = plsc.VectorSubcoreMesh(
    core_axis_name="c", subcore_axis_name="s", num_cores=NC, num_subcores=NS
)


def _bin_body(src_hbm, dst_hbm, et_hbm,
              bsrc_out, bdst_out, plen_out,
              src_v, dst_v, et_v, csrc_v, cdst_v, plen_v):
    c = lax.axis_index("c")
    s = lax.axis_index("s")
    rbase = (c * NS + s) * REG
    base = s * EPT
    iota = lax.iota(jnp.int32, LANES)
    plvec = jnp.zeros((LANES,), jnp.int32)
    off = jnp.int32(0)
    for p in range(RPC):
        rel = c * RPC + p

        def scan_chunk(kc, cnt0):
            mb = pl.ds(pl.multiple_of(base + kc * CHM, 8), CHM)
            pltpu.sync_copy(src_hbm.at[mb], src_v)
            pltpu.sync_copy(dst_hbm.at[mb], dst_v)
            pltpu.sync_copy(et_hbm.at[mb], et_v)

            def scan(i, cnt):
                sl = pl.ds(i * LANES, LANES)
                m = et_v[sl] == rel
                mi = m.astype(jnp.int32)
                pos = cnt + lax.cumsum(mi, axis=0) - mi
                plsc.store_scatter(csrc_v, [pos], src_v[sl], mask=m)
                plsc.store_scatter(cdst_v, [pos], dst_v[sl], mask=m)
                return cnt + jnp.sum(mi)

            return lax.fori_loop(0, CHM // LANES, scan, cnt0)

        cnt = lax.fori_loop(0, NCHM, scan_chunk, jnp.int32(0))
        for j in range(PADQ // LANES):
            pidx = cnt + j * LANES + iota
            plsc.store_scatter(csrc_v, [pidx], jnp.zeros((LANES,), jnp.int32))
            plsc.store_scatter(cdst_v, [pidx],
                               jnp.full((LANES,), DUMMY, jnp.int32))
        padded = ((cnt + PADQ - 1) // PADQ) * PADQ
        nout = (padded + 1023) // 1024

        def flush(k, o):
            sl_v = pl.ds(k * 1024, 1024)
            sl_h = pl.ds(pl.multiple_of(rbase + o + k * 1024, 256), 1024)
            pltpu.sync_copy(csrc_v.at[sl_v], bsrc_out.at[sl_h])
            pltpu.sync_copy(cdst_v.at[sl_v], bdst_out.at[sl_h])
            return o

        lax.fori_loop(0, nout, flush, off)
        plvec = plvec + jnp.where(iota == p, padded, 0)
        off = off + padded
    plen_v[...] = plvec
    pltpu.sync_copy(
        plen_v,
        plen_out.at[pl.ds(pl.multiple_of((c * NS + s) * LANES, LANES), LANES)])


def _bin_edges(src, dst, et):
    f = pl.kernel(
        _bin_body,
        out_type=(
            jax.ShapeDtypeStruct((NC * NS * REG,), jnp.int32),
            jax.ShapeDtypeStruct((NC * NS * REG,), jnp.int32),
            jax.ShapeDtypeStruct((NC * NS * LANES,), jnp.int32),
        ),
        mesh=_mesh,
        scratch_types=[
            pltpu.VMEM((CHM,), jnp.int32),
            pltpu.VMEM((CHM,), jnp.int32),
            pltpu.VMEM((CHM,), jnp.int32),
            pltpu.VMEM((CBUF,), jnp.int32),
            pltpu.VMEM((CBUF,), jnp.int32),
            pltpu.VMEM((LANES,), jnp.int32),
        ],
        compiler_params=pltpu.CompilerParams(needs_layout_passes=False),
    )
    return f(src, dst, et)


def _cnt_body(bdst_hbm, plens_hbm, ones_hbm, zcnt_hbm, cnt_out,
              acc_c, idx_d, ones_v, zcnt_v, plen_v):
    c = lax.axis_index("c")
    s = lax.axis_index("s")
    rbase = (c * NS + s) * REG
    iota = lax.iota(jnp.int32, LANES)
    pltpu.sync_copy(
        plens_hbm.at[pl.ds(pl.multiple_of((c * NS + s) * LANES, LANES), LANES)],
        plen_v)
    pltpu.sync_copy(ones_hbm, ones_v)
    pltpu.sync_copy(zcnt_hbm, zcnt_v)
    plvec = plen_v[...]
    row0 = pl.multiple_of(s * RPT, RPT)
    for p in range(RPC):
        for k in range(RPT // ZR):
            sl_z = pl.ds(pl.multiple_of(row0 + k * ZR, ZR), ZR)
            pltpu.sync_copy(zcnt_v, acc_c.at[sl_z])
        plsc.subcore_barrier()
        lp = jnp.max(jnp.where(iota == p, plvec, 0))
        off = jnp.sum(jnp.where(iota < p, plvec, 0))
        nch = lp // CH

        def chunk(k, carry):
            so = pl.ds(pl.multiple_of(rbase + off + k * CH, CH), CH)
            pltpu.sync_copy(bdst_hbm.at[so], idx_d)
            pltpu.sync_copy(ones_v, acc_c.at[idx_d], add=True)
            return carry

        lax.fori_loop(0, nch, chunk, 0)
        plsc.subcore_barrier()
        obase = pl.ds(pl.multiple_of((c * RPC + p) * NPAD + row0, ZR), RPT)
        pltpu.sync_copy(acc_c.at[pl.ds(row0, RPT)], cnt_out.at[obase])


def _cnt_edges(bdst, plens, ones, zcnt):
    f = pl.kernel(
        _cnt_body,
        out_type=jax.ShapeDtypeStruct((R * NPAD, H), jnp.float32),
        mesh=_mesh,
        scratch_types=[
            pltpu.VMEM_SHARED((NPAD, H), jnp.float32),
            pltpu.VMEM((CH,), jnp.int32),
            pltpu.VMEM((CH, H), jnp.float32),
            pltpu.VMEM((ZR, H), jnp.float32),
            pltpu.VMEM((LANES,), jnp.int32),
        ],
        compiler_params=pltpu.CompilerParams(needs_layout_passes=False),
    )
    return f(bdst, plens, ones, zcnt)


def _seg_body(h_hbm, bsrc_hbm, bdst_hbm, plens_hbm, zero_hbm,
              sh_out,
              acc_h, idx_s0, idx_d0, rows0, idx_s1, idx_d1, rows1, zero_v,
              plen_v, sem0, sem1, sem2, sem3):
    c = lax.axis_index("c")
    s = lax.axis_index("s")
    rbase = (c * NS + s) * REG
    iota = lax.iota(jnp.int32, LANES)
    pltpu.sync_copy(
        plens_hbm.at[pl.ds(pl.multiple_of((c * NS + s) * LANES, LANES), LANES)],
        plen_v)
    pltpu.sync_copy(zero_hbm, zero_v)
    plvec = plen_v[...]
    row0 = pl.multiple_of(s * RPT, RPT)
    bufs = ((idx_s0, idx_d0, rows0, sem0, sem2),
            (idx_s1, idx_d1, rows1, sem1, sem3))

    for p in range(RPC):
        for k in range(RPT // ZR):
            sl_z = pl.ds(pl.multiple_of(row0 + k * ZR, ZR), ZR)
            pltpu.sync_copy(zero_v, acc_h.at[sl_z])
        plsc.subcore_barrier()
        lp = jnp.max(jnp.where(iota == p, plvec, 0))
        off = jnp.sum(jnp.where(iota < p, plvec, 0))
        nch = lp // CH

        def start(k, b):
            isv, idv, rv, sg, ss = bufs[b]
            so = pl.ds(pl.multiple_of(rbase + off + k * CH, CH), CH)
            pltpu.sync_copy(bsrc_hbm.at[so], isv)
            pltpu.sync_copy(bdst_hbm.at[so], idv)
            pltpu.async_copy(h_hbm.at[isv], rv, sg)

        def wait_gather(b):
            isv, idv, rv, sg, ss = bufs[b]
            pltpu.make_async_copy(h_hbm.at[isv], rv, sg).wait()

        def fire_scatter(b):
            isv, idv, rv, sg, ss = bufs[b]
            pltpu.async_copy(rv, acc_h.at[idv], ss, add=True)

        def wait_scatter(b):
            isv, idv, rv, sg, ss = bufs[b]
            pltpu.make_async_copy(rv, acc_h.at[idv], ss).wait()

        @pl.when(nch > 0)
        def _():
            start(0, 0)

            def body(j, carry):
                k0 = 2 * j
                k1 = 2 * j + 1

                @pl.when(j >= 1)
                def _():
                    wait_scatter(1)

                start(k1, 1)
                wait_gather(0)
                fire_scatter(0)
                wait_gather(1)
                fire_scatter(1)

                @pl.when(k0 + 2 < nch)
                def _():
                    wait_scatter(0)
                    start(k0 + 2, 0)

                return carry

            lax.fori_loop(0, nch // 2, body, 0)
            wait_scatter(0)
            wait_scatter(1)

        plsc.subcore_barrier()
        obase = pl.ds(pl.multiple_of((c * RPC + p) * NPAD + row0, ZR), RPT)
        pltpu.sync_copy(acc_h.at[pl.ds(row0, RPT)], sh_out.at[obase])


def _seg_sum(h, bsrc, bdst, plens, zero):
    f = pl.kernel(
        _seg_body,
        out_type=jax.ShapeDtypeStruct((R * NPAD, H), jnp.float32),
        mesh=_mesh,
        scratch_types=[
            pltpu.VMEM_SHARED((NPAD, H), jnp.float32),
            pltpu.VMEM((CH,), jnp.int32),
            pltpu.VMEM((CH,), jnp.int32),
            pltpu.VMEM((CH, H), jnp.float32),
            pltpu.VMEM((CH,), jnp.int32),
            pltpu.VMEM((CH,), jnp.int32),
            pltpu.VMEM((CH, H), jnp.float32),
            pltpu.VMEM((ZR, H), jnp.float32),
            pltpu.VMEM((LANES,), jnp.int32),
            pltpu.SemaphoreType.DMA,
            pltpu.SemaphoreType.DMA,
            pltpu.SemaphoreType.DMA,
            pltpu.SemaphoreType.DMA,
        ],
        compiler_params=pltpu.CompilerParams(needs_layout_passes=False),
    )
    return f(h, bsrc, bdst, plens, zero)


def _matmul_body(x_ref, w_ref, b_ref, o_ref):
    o_ref[...] = (
        jnp.dot(x_ref[...], w_ref[...], preferred_element_type=jnp.float32)
        + b_ref[...]
    )


def _in_proj(x, w, b):
    return pl.pallas_call(
        _matmul_body,
        grid=(GRID,),
        in_specs=[
            pl.BlockSpec((BN, H), lambda i: (i, 0)),
            pl.BlockSpec((H, H), lambda i: (0, 0)),
            pl.BlockSpec((1, H), lambda i: (0, 0)),
        ],
        out_specs=pl.BlockSpec((BN, H), lambda i: (i, 0)),
        out_shape=jax.ShapeDtypeStruct((N, H), jnp.float32),
    )(x, w, b)


def _dense_body(sh_ref, scnt_ref, h_ref, w_ref, wroot_ref, b_ref,
                hpre_ref, stats_ref, acc_ref):
    i = pl.program_id(0)
    agg = jnp.zeros((BN, H), jnp.float32)
    for r in range(R):
        cnt = scnt_ref[:, r:r + 1]
        rc = 1.0 / jnp.maximum(cnt, 1.0)
        agg = agg + jnp.dot(sh_ref[r] * rc, w_ref[r],
                            preferred_element_type=jnp.float32)
    agg = agg + jnp.dot(h_ref[...], wroot_ref[...],
                        preferred_element_type=jnp.float32) + b_ref[...]

    @pl.when(i == 0)
    def _():
        acc_ref[...] = jnp.zeros((2, H), jnp.float32)

    colsum = jnp.sum(agg, axis=0)[None, :]
    colsq = jnp.sum(agg * agg, axis=0)[None, :]
    acc_ref[...] = acc_ref[...] + jnp.concatenate([colsum, colsq], axis=0)

    @pl.when(i == pl.num_programs(0) - 1)
    def _():
        stats_ref[...] = acc_ref[...]

    hpre_ref[...] = agg


def _dense(sh, scnt, h, w, wroot, b):
    return pl.pallas_call(
        _dense_body,
        grid=(GRID,),
        in_specs=[
            pl.BlockSpec((R, BN, H), lambda i: (0, i, 0)),
            pl.BlockSpec((BN, R), lambda i: (i, 0)),
            pl.BlockSpec((BN, H), lambda i: (i, 0)),
            pl.BlockSpec((R, H, H), lambda i: (0, 0, 0)),
            pl.BlockSpec((H, H), lambda i: (0, 0)),
            pl.BlockSpec((1, H), lambda i: (0, 0)),
        ],
        out_specs=[
            pl.BlockSpec((BN, H), lambda i: (i, 0)),
            pl.BlockSpec((2, H), lambda i: (0, 0)),
        ],
        out_shape=[
            jax.ShapeDtypeStruct((N, H), jnp.float32),
            jax.ShapeDtypeStruct((2, H), jnp.float32),
        ],
        scratch_shapes=[pltpu.VMEM((2, H), jnp.float32)],
    )(sh, scnt, h, w, wroot, b)


def _bn_body(hpre_ref, stats_ref, g_ref, b_ref, o_ref):
    mu = stats_ref[0:1, :] * (1.0 / N)
    ex2 = stats_ref[1:2, :] * (1.0 / N)
    var = ex2 - mu * mu
    inv = lax.rsqrt(var + 1e-5)
    y = g_ref[...] * (hpre_ref[...] - mu) * inv + b_ref[...]
    o_ref[...] = jnp.where(y >= 0.0, y, 0.1 * y)


def _bn_apply(hpre, stats, g, b):
    return pl.pallas_call(
        _bn_body,
        grid=(GRID,),
        in_specs=[
            pl.BlockSpec((BN, H), lambda i: (i, 0)),
            pl.BlockSpec((2, H), lambda i: (0, 0)),
            pl.BlockSpec((1, H), lambda i: (0, 0)),
            pl.BlockSpec((1, H), lambda i: (0, 0)),
        ],
        out_specs=pl.BlockSpec((BN, H), lambda i: (i, 0)),
        out_shape=jax.ShapeDtypeStruct((N, H), jnp.float32),
    )(hpre, stats, g, b)


def _fuse_body(h1_ref, h2_ref, h3_ref, w1_ref, b1_ref, w2_ref, b2_ref,
               ow_ref, ob_ref, o_ref):
    hs = (h1_ref[...], h2_ref[...], h3_ref[...])
    ss = []
    for hr in hs:
        t = jnp.maximum(
            jnp.dot(hr, w1_ref[...], preferred_element_type=jnp.float32)
            + b1_ref[...], 0.0)
        ss.append(jnp.sum(t * w2_ref[...], axis=1, keepdims=True)
                  + b2_ref[...])
    m = jnp.maximum(jnp.maximum(ss[0], ss[1]), ss[2])
    es = [jnp.exp(s - m) for s in ss]
    z = es[0] + es[1] + es[2]
    ms = (es[0] * hs[0] + es[1] * hs[1] + es[2] * hs[2]) / z
    out = jnp.dot(ms, ow_ref[...], preferred_element_type=jnp.float32) \
        + ob_ref[...]
    nrm = jnp.sqrt(jnp.sum(out * out, axis=1, keepdims=True))
    o_ref[...] = out / jnp.maximum(nrm, 1e-12)


def _fuse(h1, h2, h3, w1, b1, w2t, b2, ow, ob):
    hq = H // 4
    return pl.pallas_call(
        _fuse_body,
        grid=(GRID,),
        in_specs=[
            pl.BlockSpec((BN, H), lambda i: (i, 0)),
            pl.BlockSpec((BN, H), lambda i: (i, 0)),
            pl.BlockSpec((BN, H), lambda i: (i, 0)),
            pl.BlockSpec((H, hq), lambda i: (0, 0)),
            pl.BlockSpec((1, hq), lambda i: (0, 0)),
            pl.BlockSpec((1, hq), lambda i: (0, 0)),
            pl.BlockSpec((1, 1), lambda i: (0, 0)),
            pl.BlockSpec((H, H), lambda i: (0, 0)),
            pl.BlockSpec((1, H), lambda i: (0, 0)),
        ],
        out_specs=pl.BlockSpec((BN, H), lambda i: (i, 0)),
        out_shape=jax.ShapeDtypeStruct((N, H), jnp.float32),
    )(h1, h2, h3, w1, b1, w2t, b2, ow, ob)


def kernel(x, edge_index, edge_type, params):
    src = edge_index[0]
    dst = edge_index[1]
    et = edge_type.astype(jnp.int32)

    zero = jnp.zeros((ZR, H), jnp.float32)
    ones = jnp.ones((CH, H), jnp.float32)

    bsrc, bdst, plens = _bin_edges(src, dst, et)
    cnt = _cnt_edges(bdst, plens, ones, zero)
    scnt = cnt.reshape(R, NPAD, H)[:, :, 0].T

    h = _in_proj(x, params['in_w'], params['in_b'].reshape(1, H))
    reps = []
    for i in range(L):
        sh = _seg_sum(h, bsrc, bdst, plens, zero)
        sh = sh.reshape(R, NPAD, H)
        hpre, stats = _dense(sh, scnt, h, params['conv_w'][i],
                             params['conv_root'][i],
                             params['conv_b'][i].reshape(1, H))
        h = _bn_apply(hpre, stats, params['bn_g'][i].reshape(1, H),
                      params['bn_b'][i].reshape(1, H))
        reps.append(h)

    out = _fuse(
        reps[0], reps[1], reps[2],
        params['att_w1'], params['att_b1'].reshape(1, H // 4),
        params['att_w2'].reshape(1, H // 4), params['att_b2'].reshape(1, 1),
        params['out_w'], params['out_b'].reshape(1, H),
    )
    return out


# speedup vs baseline: 9.4199x; 1.1329x over previous
import jax
import jax.numpy as jnp
from jax import lax
from jax.experimental import pallas as pl
from jax.experimental.pallas import tpu as pltpu
from jax.experimental.pallas import tpu_sc as plsc

N = 10000
E = 320000
H = 128
R = 8
L = 3
NC = 2
NS = 16
LANES = 16
RPC = R // NC
EPT = E // NS
NVEC = EPT // LANES
CH = 128
PADQ = 256
REG = 22528
CBUF = 20608
CHM = 400
NCHM = EPT // CHM
NPAD = 10240
RPT = NPAD // NS
DUMMY = 10200
ZR = 64
BN = 1000
GRID = N // BN

_mesh = plsc.VectorSubcoreMesh(
    core_axis_name="c", subcore_axis_name="s", num_cores=NC, num_subcores=NS
)


def _bin_body(src_hbm, dst_hbm, et_hbm,
              bsrc_out, bdst_out, plen_out,
              src_v0, dst_v0, et_v0, src_v1, dst_v1, et_v1,
              csrc_v, cdst_v, plen_v, msem0, msem1):
    c = lax.axis_index("c")
    s = lax.axis_index("s")
    rbase = (c * NS + s) * REG
    base = s * EPT
    iota = lax.iota(jnp.int32, LANES)
    mbufs = ((src_v0, dst_v0, et_v0, msem0), (src_v1, dst_v1, et_v1, msem1))

    def mload(kc, b):
        sv, dv, ev, sm = mbufs[b]
        mb = pl.ds(pl.multiple_of(base + kc * CHM, 8), CHM)
        pltpu.async_copy(src_hbm.at[mb], sv, sm)
        pltpu.async_copy(dst_hbm.at[mb], dv, sm)
        pltpu.async_copy(et_hbm.at[mb], ev, sm)

    def mwait(kc, b):
        sv, dv, ev, sm = mbufs[b]
        mb = pl.ds(pl.multiple_of(base + kc * CHM, 8), CHM)
        pltpu.make_async_copy(src_hbm.at[mb], sv, sm).wait()
        pltpu.make_async_copy(dst_hbm.at[mb], dv, sm).wait()
        pltpu.make_async_copy(et_hbm.at[mb], ev, sm).wait()

    plvec = jnp.zeros((LANES,), jnp.int32)
    off = jnp.int32(0)
    for p in range(RPC):
        rel = c * RPC + p

        def scan_buf(b, cnt0):
            sv, dv, ev, sm = mbufs[b]

            def scan(i, cnt):
                sl = pl.ds(i * LANES, LANES)
                m = ev[sl] == rel
                mi = m.astype(jnp.int32)
                pos = cnt + lax.cumsum(mi, axis=0) - mi
                plsc.store_scatter(csrc_v, [pos], sv[sl], mask=m)
                plsc.store_scatter(cdst_v, [pos], dv[sl], mask=m)
                return cnt + jnp.sum(mi)

            return lax.fori_loop(0, CHM // LANES, scan, cnt0)

        def pair(j, cnt0):
            k0 = 2 * j
            k1 = 2 * j + 1
            mload(k1, 1)
            mwait(k0, 0)
            cnt1 = scan_buf(0, cnt0)

            @pl.when(k1 + 1 < NCHM)
            def _():
                mload(k1 + 1, 0)

            mwait(k1, 1)
            return scan_buf(1, cnt1)

        mload(0, 0)
        cnt = lax.fori_loop(0, NCHM // 2, pair, jnp.int32(0))
        for j in range(PADQ // LANES):
            pidx = cnt + j * LANES + iota
            plsc.store_scatter(csrc_v, [pidx], jnp.zeros((LANES,), jnp.int32))
            plsc.store_scatter(cdst_v, [pidx],
                               jnp.full((LANES,), DUMMY, jnp.int32))
        padded = ((cnt + PADQ - 1) // PADQ) * PADQ
        nout = (padded + 1023) // 1024

        def flush(k, o):
            sl_v = pl.ds(k * 1024, 1024)
            sl_h = pl.ds(pl.multiple_of(rbase + o + k * 1024, 256), 1024)
            pltpu.sync_copy(csrc_v.at[sl_v], bsrc_out.at[sl_h])
            pltpu.sync_copy(cdst_v.at[sl_v], bdst_out.at[sl_h])
            return o

        lax.fori_loop(0, nout, flush, off)
        plvec = plvec + jnp.where(iota == p, padded, 0)
        off = off + padded
    plen_v[...] = plvec
    pltpu.sync_copy(
        plen_v,
        plen_out.at[pl.ds(pl.multiple_of((c * NS + s) * LANES, LANES), LANES)])


def _bin_edges(src, dst, et):
    f = pl.kernel(
        _bin_body,
        out_type=(
            jax.ShapeDtypeStruct((NC * NS * REG,), jnp.int32),
            jax.ShapeDtypeStruct((NC * NS * REG,), jnp.int32),
            jax.ShapeDtypeStruct((NC * NS * LANES,), jnp.int32),
        ),
        mesh=_mesh,
        scratch_types=[
            pltpu.VMEM((CHM,), jnp.int32),
            pltpu.VMEM((CHM,), jnp.int32),
            pltpu.VMEM((CHM,), jnp.int32),
            pltpu.VMEM((CHM,), jnp.int32),
            pltpu.VMEM((CHM,), jnp.int32),
            pltpu.VMEM((CHM,), jnp.int32),
            pltpu.VMEM((CBUF,), jnp.int32),
            pltpu.VMEM((CBUF,), jnp.int32),
            pltpu.VMEM((LANES,), jnp.int32),
            pltpu.SemaphoreType.DMA,
            pltpu.SemaphoreType.DMA,
        ],
        compiler_params=pltpu.CompilerParams(needs_layout_passes=False),
    )
    return f(src, dst, et)


def _cnt_body(bdst_hbm, plens_hbm, ones_hbm, zcnt_hbm, cnt_out,
              acc_c, idx_d, ones_v, zcnt_v, plen_v):
    c = lax.axis_index("c")
    s = lax.axis_index("s")
    rbase = (c * NS + s) * REG
    iota = lax.iota(jnp.int32, LANES)
    pltpu.sync_copy(
        plens_hbm.at[pl.ds(pl.multiple_of((c * NS + s) * LANES, LANES), LANES)],
        plen_v)
    pltpu.sync_copy(ones_hbm, ones_v)
    pltpu.sync_copy(zcnt_hbm, zcnt_v)
    plvec = plen_v[...]
    row0 = pl.multiple_of(s * RPT, RPT)
    for p in range(RPC):
        for k in range(RPT // ZR):
            sl_z = pl.ds(pl.multiple_of(row0 + k * ZR, ZR), ZR)
            pltpu.sync_copy(zcnt_v, acc_c.at[sl_z])
        plsc.subcore_barrier()
        lp = jnp.max(jnp.where(iota == p, plvec, 0))
        off = jnp.sum(jnp.where(iota < p, plvec, 0))
        nch = lp // CH

        def chunk(k, carry):
            so = pl.ds(pl.multiple_of(rbase + off + k * CH, CH), CH)
            pltpu.sync_copy(bdst_hbm.at[so], idx_d)
            pltpu.sync_copy(ones_v, acc_c.at[idx_d], add=True)
            return carry

        lax.fori_loop(0, nch, chunk, 0)
        plsc.subcore_barrier()
        obase = pl.ds(pl.multiple_of((c * RPC + p) * NPAD + row0, ZR), RPT)
        pltpu.sync_copy(acc_c.at[pl.ds(row0, RPT)], cnt_out.at[obase])


def _cnt_edges(bdst, plens, ones, zcnt):
    f = pl.kernel(
        _cnt_body,
        out_type=jax.ShapeDtypeStruct((R * NPAD, H), jnp.float32),
        mesh=_mesh,
        scratch_types=[
            pltpu.VMEM_SHARED((NPAD, H), jnp.float32),
            pltpu.VMEM((CH,), jnp.int32),
            pltpu.VMEM((CH, H), jnp.float32),
            pltpu.VMEM((ZR, H), jnp.float32),
            pltpu.VMEM((LANES,), jnp.int32),
        ],
        compiler_params=pltpu.CompilerParams(needs_layout_passes=False),
    )
    return f(bdst, plens, ones, zcnt)


def _seg_body(h_hbm, bsrc_hbm, bdst_hbm, plens_hbm, zero_hbm,
              sh_out,
              acc_h, idx_s0, idx_d0, rows0, idx_s1, idx_d1, rows1, zero_v,
              plen_v, sem0, sem1, sem2, sem3):
    c = lax.axis_index("c")
    s = lax.axis_index("s")
    rbase = (c * NS + s) * REG
    iota = lax.iota(jnp.int32, LANES)
    pltpu.sync_copy(
        plens_hbm.at[pl.ds(pl.multiple_of((c * NS + s) * LANES, LANES), LANES)],
        plen_v)
    pltpu.sync_copy(zero_hbm, zero_v)
    plvec = plen_v[...]
    row0 = pl.multiple_of(s * RPT, RPT)
    bufs = ((idx_s0, idx_d0, rows0, sem0, sem2),
            (idx_s1, idx_d1, rows1, sem1, sem3))

    for p in range(RPC):
        for k in range(RPT // ZR):
            sl_z = pl.ds(pl.multiple_of(row0 + k * ZR, ZR), ZR)
            pltpu.sync_copy(zero_v, acc_h.at[sl_z])
        plsc.subcore_barrier()
        lp = jnp.max(jnp.where(iota == p, plvec, 0))
        off = jnp.sum(jnp.where(iota < p, plvec, 0))
        nch = lp // CH

        def start(k, b):
            isv, idv, rv, sg, ss = bufs[b]
            so = pl.ds(pl.multiple_of(rbase + off + k * CH, CH), CH)
            pltpu.sync_copy(bsrc_hbm.at[so], isv)
            pltpu.sync_copy(bdst_hbm.at[so], idv)
            pltpu.async_copy(h_hbm.at[isv], rv, sg)

        def wait_gather(b):
            isv, idv, rv, sg, ss = bufs[b]
            pltpu.make_async_copy(h_hbm.at[isv], rv, sg).wait()

        def fire_scatter(b):
            isv, idv, rv, sg, ss = bufs[b]
            pltpu.async_copy(rv, acc_h.at[idv], ss, add=True)

        def wait_scatter(b):
            isv, idv, rv, sg, ss = bufs[b]
            pltpu.make_async_copy(rv, acc_h.at[idv], ss).wait()

        @pl.when(nch > 0)
        def _():
            start(0, 0)

            def body(j, carry):
                k0 = 2 * j
                k1 = 2 * j + 1

                @pl.when(j >= 1)
                def _():
                    wait_scatter(1)

                start(k1, 1)
                wait_gather(0)
                fire_scatter(0)
                wait_gather(1)
                fire_scatter(1)

                @pl.when(k0 + 2 < nch)
                def _():
                    wait_scatter(0)
                    start(k0 + 2, 0)

                return carry

            lax.fori_loop(0, nch // 2, body, 0)
            wait_scatter(0)
            wait_scatter(1)

        plsc.subcore_barrier()
        obase = pl.ds(pl.multiple_of((c * RPC + p) * NPAD + row0, ZR), RPT)
        pltpu.sync_copy(acc_h.at[pl.ds(row0, RPT)], sh_out.at[obase])


def _seg_sum(h, bsrc, bdst, plens, zero):
    f = pl.kernel(
        _seg_body,
        out_type=jax.ShapeDtypeStruct((R * NPAD, H), jnp.float32),
        mesh=_mesh,
        scratch_types=[
            pltpu.VMEM_SHARED((NPAD, H), jnp.float32),
            pltpu.VMEM((CH,), jnp.int32),
            pltpu.VMEM((CH,), jnp.int32),
            pltpu.VMEM((CH, H), jnp.float32),
            pltpu.VMEM((CH,), jnp.int32),
            pltpu.VMEM((CH,), jnp.int32),
            pltpu.VMEM((CH, H), jnp.float32),
            pltpu.VMEM((ZR, H), jnp.float32),
            pltpu.VMEM((LANES,), jnp.int32),
            pltpu.SemaphoreType.DMA,
            pltpu.SemaphoreType.DMA,
            pltpu.SemaphoreType.DMA,
            pltpu.SemaphoreType.DMA,
        ],
        compiler_params=pltpu.CompilerParams(needs_layout_passes=False),
    )
    return f(h, bsrc, bdst, plens, zero)


def _matmul_body(x_ref, w_ref, b_ref, o_ref):
    o_ref[...] = (
        jnp.dot(x_ref[...], w_ref[...], preferred_element_type=jnp.float32)
        + b_ref[...]
    )


def _in_proj(x, w, b):
    return pl.pallas_call(
        _matmul_body,
        grid=(GRID,),
        in_specs=[
            pl.BlockSpec((BN, H), lambda i: (i, 0)),
            pl.BlockSpec((H, H), lambda i: (0, 0)),
            pl.BlockSpec((1, H), lambda i: (0, 0)),
        ],
        out_specs=pl.BlockSpec((BN, H), lambda i: (i, 0)),
        out_shape=jax.ShapeDtypeStruct((N, H), jnp.float32),
    )(x, w, b)


def _dense_body(sh_ref, scnt_ref, h_ref, w_ref, wroot_ref, b_ref,
                hpre_ref, stats_ref, acc_ref):
    i = pl.program_id(0)
    agg = jnp.zeros((BN, H), jnp.float32)
    for r in range(R):
        cnt = scnt_ref[:, r:r + 1]
        rc = 1.0 / jnp.maximum(cnt, 1.0)
        agg = agg + jnp.dot(sh_ref[r] * rc, w_ref[r],
                            preferred_element_type=jnp.float32)
    agg = agg + jnp.dot(h_ref[...], wroot_ref[...],
                        preferred_element_type=jnp.float32) + b_ref[...]

    @pl.when(i == 0)
    def _():
        acc_ref[...] = jnp.zeros((2, H), jnp.float32)

    colsum = jnp.sum(agg, axis=0)[None, :]
    colsq = jnp.sum(agg * agg, axis=0)[None, :]
    acc_ref[...] = acc_ref[...] + jnp.concatenate([colsum, colsq], axis=0)

    @pl.when(i == pl.num_programs(0) - 1)
    def _():
        stats_ref[...] = acc_ref[...]

    hpre_ref[...] = agg


def _dense(sh, scnt, h, w, wroot, b):
    return pl.pallas_call(
        _dense_body,
        grid=(GRID,),
        in_specs=[
            pl.BlockSpec((R, BN, H), lambda i: (0, i, 0)),
            pl.BlockSpec((BN, R), lambda i: (i, 0)),
            pl.BlockSpec((BN, H), lambda i: (i, 0)),
            pl.BlockSpec((R, H, H), lambda i: (0, 0, 0)),
            pl.BlockSpec((H, H), lambda i: (0, 0)),
            pl.BlockSpec((1, H), lambda i: (0, 0)),
        ],
        out_specs=[
            pl.BlockSpec((BN, H), lambda i: (i, 0)),
            pl.BlockSpec((2, H), lambda i: (0, 0)),
        ],
        out_shape=[
            jax.ShapeDtypeStruct((N, H), jnp.float32),
            jax.ShapeDtypeStruct((2, H), jnp.float32),
        ],
        scratch_shapes=[pltpu.VMEM((2, H), jnp.float32)],
    )(sh, scnt, h, w, wroot, b)


def _bn_body(hpre_ref, stats_ref, g_ref, b_ref, o_ref):
    mu = stats_ref[0:1, :] * (1.0 / N)
    ex2 = stats_ref[1:2, :] * (1.0 / N)
    var = ex2 - mu * mu
    inv = lax.rsqrt(var + 1e-5)
    y = g_ref[...] * (hpre_ref[...] - mu) * inv + b_ref[...]
    o_ref[...] = jnp.where(y >= 0.0, y, 0.1 * y)


def _bn_apply(hpre, stats, g, b):
    return pl.pallas_call(
        _bn_body,
        grid=(GRID,),
        in_specs=[
            pl.BlockSpec((BN, H), lambda i: (i, 0)),
            pl.BlockSpec((2, H), lambda i: (0, 0)),
            pl.BlockSpec((1, H), lambda i: (0, 0)),
            pl.BlockSpec((1, H), lambda i: (0, 0)),
        ],
        out_specs=pl.BlockSpec((BN, H), lambda i: (i, 0)),
        out_shape=jax.ShapeDtypeStruct((N, H), jnp.float32),
    )(hpre, stats, g, b)


def _fuse_body(h1_ref, h2_ref, h3_ref, w1_ref, b1_ref, w2_ref, b2_ref,
               ow_ref, ob_ref, o_ref):
    hs = (h1_ref[...], h2_ref[...], h3_ref[...])
    ss = []
    for hr in hs:
        t = jnp.maximum(
            jnp.dot(hr, w1_ref[...], preferred_element_type=jnp.float32)
            + b1_ref[...], 0.0)
        ss.append(jnp.sum(t * w2_ref[...], axis=1, keepdims=True)
                  + b2_ref[...])
    m = jnp.maximum(jnp.maximum(ss[0], ss[1]), ss[2])
    es = [jnp.exp(s - m) for s in ss]
    z = es[0] + es[1] + es[2]
    ms = (es[0] * hs[0] + es[1] * hs[1] + es[2] * hs[2]) / z
    out = jnp.dot(ms, ow_ref[...], preferred_element_type=jnp.float32) \
        + ob_ref[...]
    nrm = jnp.sqrt(jnp.sum(out * out, axis=1, keepdims=True))
    o_ref[...] = out / jnp.maximum(nrm, 1e-12)


def _fuse(h1, h2, h3, w1, b1, w2t, b2, ow, ob):
    hq = H // 4
    return pl.pallas_call(
        _fuse_body,
        grid=(GRID,),
        in_specs=[
            pl.BlockSpec((BN, H), lambda i: (i, 0)),
            pl.BlockSpec((BN, H), lambda i: (i, 0)),
            pl.BlockSpec((BN, H), lambda i: (i, 0)),
            pl.BlockSpec((H, hq), lambda i: (0, 0)),
            pl.BlockSpec((1, hq), lambda i: (0, 0)),
            pl.BlockSpec((1, hq), lambda i: (0, 0)),
            pl.BlockSpec((1, 1), lambda i: (0, 0)),
            pl.BlockSpec((H, H), lambda i: (0, 0)),
            pl.BlockSpec((1, H), lambda i: (0, 0)),
        ],
        out_specs=pl.BlockSpec((BN, H), lambda i: (i, 0)),
        out_shape=jax.ShapeDtypeStruct((N, H), jnp.float32),
    )(h1, h2, h3, w1, b1, w2t, b2, ow, ob)


def kernel(x, edge_index, edge_type, params):
    src = edge_index[0]
    dst = edge_index[1]
    et = edge_type.astype(jnp.int32)

    zero = jnp.zeros((ZR, H), jnp.float32)
    ones = jnp.ones((CH, H), jnp.float32)

    bsrc, bdst, plens = _bin_edges(src, dst, et)
    cnt = _cnt_edges(bdst, plens, ones, zero)
    scnt = cnt.reshape(R, NPAD, H)[:, :, 0].T

    h = _in_proj(x, params['in_w'], params['in_b'].reshape(1, H))
    reps = []
    for i in range(L):
        sh = _seg_sum(h, bsrc, bdst, plens, zero)
        sh = sh.reshape(R, NPAD, H)
        hpre, stats = _dense(sh, scnt, h, params['conv_w'][i],
                             params['conv_root'][i],
                             params['conv_b'][i].reshape(1, H))
        h = _bn_apply(hpre, stats, params['bn_g'][i].reshape(1, H),
                      params['bn_b'][i].reshape(1, H))
        reps.append(h)

    out = _fuse(
        reps[0], reps[1], reps[2],
        params['att_w1'], params['att_b1'].reshape(1, H // 4),
        params['att_w2'].reshape(1, H // 4), params['att_b2'].reshape(1, 1),
        params['out_w'], params['out_b'].reshape(1, H),
    )
    return out
